# Initial kernel scaffold; baseline (speedup 1.0000x reference)
import jax
import jax.numpy as jnp
from jax.experimental import pallas as pl


def kernel(token, row_embed, col_embed):
    raise NotImplementedError("write your pallas kernel here")


# speedup vs baseline: 1.9950x; 1.9950x over previous
import jax
import jax.numpy as jnp
from jax import lax
from jax.experimental import pallas as pl
from jax.experimental.pallas import tpu as pltpu
from jax.experimental.pallas import tpu_sc as plsc

H, W = 512, 512
HALF = 384
B, L = 1024, 50
NC, NS = 2, 16
NW = NC * NS
ROWS = B * L * 2
ROWS_PER_W = ROWS // NW
CHUNK = 128
NCHUNK = ROWS_PER_W // CHUNK


def _sc_gather(table, tok):
    mesh = plsc.VectorSubcoreMesh(core_axis_name="c", subcore_axis_name="s")

    @pl.kernel(
        out_type=jax.ShapeDtypeStruct((ROWS, HALF), jnp.float32),
        mesh=mesh,
        scratch_types=[
            pltpu.VMEM((NCHUNK, CHUNK), jnp.int32),
            pltpu.VMEM((2, CHUNK, HALF), jnp.float32),
            pltpu.SemaphoreType.DMA,
            pltpu.SemaphoreType.DMA,
            pltpu.SemaphoreType.DMA,
            pltpu.SemaphoreType.DMA,
        ],
    )
    def k(table_hbm, tok_hbm, out_hbm, idx_v, buf_v, g0, g1, s0, s1):
        wid = lax.axis_index("s") * NC + lax.axis_index("c")
        base = wid * ROWS_PER_W

        pltpu.sync_copy(tok_hbm.at[wid], idx_v)
        offs = (lax.rem(lax.iota(jnp.int32, 16), 2)) * 512
        for j in range(NCHUNK):
            for q in range(CHUNK // 16):
                sl = pl.ds(q * 16, 16)
                idx_v[j, sl] = idx_v[j, sl] + offs

        gsem = (g0, g1)
        ssem = (s0, s1)
        h_g = [None] * NCHUNK
        h_s = [None] * NCHUNK
        h_g[0] = pltpu.async_copy(table_hbm.at[idx_v.at[0]], buf_v.at[0],
                                  gsem[0])
        for j in range(NCHUNK):
            cur = j % 2
            nxt = (j + 1) % 2
            if j + 1 < NCHUNK:
                if j >= 1:
                    h_s[j - 1].wait()
                h_g[j + 1] = pltpu.async_copy(
                    table_hbm.at[idx_v.at[j + 1]], buf_v.at[nxt], gsem[nxt])
            h_g[j].wait()
            h_s[j] = pltpu.async_copy(
                buf_v.at[cur],
                out_hbm.at[pl.ds(base + j * CHUNK, CHUNK)],
                ssem[cur])
        h_s[NCHUNK - 2].wait()
        h_s[NCHUNK - 1].wait()

    return k(table, tok)


def kernel(token, row_embed, col_embed):
    table = jnp.concatenate([row_embed, col_embed], axis=0)
    tok = token.astype(jnp.int32).reshape(NW, NCHUNK, CHUNK)
    out = _sc_gather(table, tok)
    return out.reshape(B, L, 2 * HALF)


# speedup vs baseline: 3.3181x; 1.6633x over previous
import jax
import jax.numpy as jnp
from jax import lax
from jax.experimental import pallas as pl
from jax.experimental.pallas import tpu as pltpu
from jax.experimental.pallas import tpu_sc as plsc

HALF = 384
B, L = 1024, 50
NC, NS = 2, 16
NW = NC * NS
B_PER_W = B // NW
L_PAD = 56
IDX_PER_W = B_PER_W * L_PAD


def _sc_gather(row_embed, col_embed, tok):
    mesh = plsc.VectorSubcoreMesh(core_axis_name="c", subcore_axis_name="s")

    @pl.kernel(
        out_type=jax.ShapeDtypeStruct((B, L, 2 * HALF), jnp.float32),
        mesh=mesh,
        scratch_types=[
            pltpu.VMEM((2 * IDX_PER_W,), jnp.int32),
            pltpu.VMEM((2, 2, L, HALF), jnp.float32),
            pltpu.SemaphoreType.DMA,
            pltpu.SemaphoreType.DMA,
            pltpu.SemaphoreType.DMA,
            pltpu.SemaphoreType.DMA,
        ],
    )
    def k(row_hbm, col_hbm, tok_hbm, out_hbm, tok_v, buf_v, g0, g1, s0, s1):
        wid = lax.axis_index("s") * NC + lax.axis_index("c")
        wb = wid * B_PER_W

        pltpu.sync_copy(tok_hbm.at[0, wid], tok_v.at[pl.ds(0, IDX_PER_W)])
        pltpu.sync_copy(tok_hbm.at[1, wid],
                        tok_v.at[pl.ds(IDX_PER_W, IDX_PER_W)])

        def idx(half, j):
            return tok_v.at[pl.ds(half * IDX_PER_W + j * L_PAD, L)]

        gsem = (g0, g1)
        ssem = (s0, s1)

        def start_gathers(j, k_, sem):
            hy = pltpu.async_copy(row_hbm.at[idx(0, j)], buf_v.at[k_, 0], sem)
            hx = pltpu.async_copy(col_hbm.at[idx(1, j)], buf_v.at[k_, 1], sem)
            return hy, hx

        h_g = [None] * B_PER_W
        h_s = [None] * B_PER_W
        h_g[0] = start_gathers(0, 0, gsem[0])
        for j in range(B_PER_W):
            cur = j % 2
            nxt = (j + 1) % 2
            if j + 1 < B_PER_W:
                if j >= 1:
                    for h in h_s[j - 1]:
                        h.wait()
                h_g[j + 1] = start_gathers(j + 1, nxt, gsem[nxt])
            for h in h_g[j]:
                h.wait()
            h_s[j] = (
                pltpu.async_copy(buf_v.at[cur, 0],
                                 out_hbm.at[wb + j, :, pl.ds(0, HALF)],
                                 ssem[cur]),
                pltpu.async_copy(buf_v.at[cur, 1],
                                 out_hbm.at[wb + j, :, pl.ds(HALF, HALF)],
                                 ssem[cur]),
            )
        for j in (B_PER_W - 2, B_PER_W - 1):
            for h in h_s[j]:
                h.wait()

    return k(row_embed, col_embed, tok)


def kernel(token, row_embed, col_embed):
    tok = jnp.pad(token.astype(jnp.int32).transpose(2, 0, 1),
                  ((0, 0), (0, 0), (0, L_PAD - L)))
    tok = tok.reshape(2, NW, IDX_PER_W)
    return _sc_gather(row_embed, col_embed, tok)


# speedup vs baseline: 5.7280x; 1.7263x over previous
import jax
import jax.numpy as jnp
from jax import lax
from jax.experimental import pallas as pl
from jax.experimental.pallas import tpu as pltpu
from jax.experimental.pallas import tpu_sc as plsc

HALF = 384
B, L = 1024, 50
NC, NS = 2, 16
NW = NC * NS
BB = B // 128
CPW = 2 * L * BB // NW


def _sc_gather(table, tok):
    mesh = plsc.VectorSubcoreMesh(core_axis_name="c", subcore_axis_name="s")

    @pl.kernel(
        out_type=jax.ShapeDtypeStruct((L, B, 2 * HALF), jnp.float32),
        mesh=mesh,
        scratch_types=[
            pltpu.VMEM((CPW, 128), jnp.int32),
            pltpu.VMEM((2, 128, HALF), jnp.float32),
            pltpu.SemaphoreType.DMA,
            pltpu.SemaphoreType.DMA,
            pltpu.SemaphoreType.DMA,
            pltpu.SemaphoreType.DMA,
        ],
    )
    def k(table_hbm, tok_hbm, out_hbm, idx_v, buf_v, g0, g1, s0, s1):
        wid = lax.axis_index("c") * NS + lax.axis_index("s")
        half = wid // NS
        c0 = wid * CPW

        pltpu.sync_copy(tok_hbm.at[wid], idx_v)
        bias = jnp.full((16,), half * 512, dtype=jnp.int32)
        for i in range(CPW):
            for q in range(8):
                sl = pl.ds(q * 16, 16)
                idx_v[i, sl] = idx_v[i, sl] + bias

        def dst(i):
            r = c0 + i - half * (L * BB)
            l = r // BB
            bb = r % BB
            return out_hbm.at[l, pl.ds(bb * 128, 128),
                              pl.ds(half * HALF, HALF)]

        gsem = (g0, g1)
        ssem = (s0, s1)
        h_g = [None] * CPW
        h_s = [None] * CPW
        h_g[0] = pltpu.async_copy(table_hbm.at[idx_v.at[0]], buf_v.at[0],
                                  gsem[0])
        for i in range(CPW):
            cur = i % 2
            nxt = (i + 1) % 2
            if i + 1 < CPW:
                if i >= 1:
                    h_s[i - 1].wait()
                h_g[i + 1] = pltpu.async_copy(
                    table_hbm.at[idx_v.at[i + 1]], buf_v.at[nxt], gsem[nxt])
            h_g[i].wait()
            h_s[i] = pltpu.async_copy(buf_v.at[cur], dst(i), ssem[cur])
        h_s[CPW - 2].wait()
        h_s[CPW - 1].wait()

    return k(table, tok)


def kernel(token, row_embed, col_embed):
    table = jnp.concatenate([row_embed, col_embed], axis=0)
    tok = token.astype(jnp.int32).transpose(2, 1, 0).reshape(NW, CPW, 128)
    out = _sc_gather(table, tok)
    return out.transpose(1, 0, 2)


# speedup vs baseline: 5.7437x; 1.0027x over previous
import jax
import jax.numpy as jnp
from jax import lax
from jax.experimental import pallas as pl
from jax.experimental.pallas import tpu as pltpu
from jax.experimental.pallas import tpu_sc as plsc

HALF = 384
B, L = 1024, 50
NC, NS = 2, 16
NW = NC * NS
BB = B // 128
CPW = 2 * L * BB // NW


def _sc_gather(table, tok):
    mesh = plsc.VectorSubcoreMesh(core_axis_name="c", subcore_axis_name="s")

    @pl.kernel(
        out_type=jax.ShapeDtypeStruct((L, B, 2 * HALF), jnp.float32),
        mesh=mesh,
        scratch_types=[
            pltpu.VMEM((CPW, 128), jnp.int32),
            pltpu.VMEM((4, 64, HALF), jnp.float32),
            pltpu.SemaphoreType.DMA,
            pltpu.SemaphoreType.DMA,
            pltpu.SemaphoreType.DMA,
            pltpu.SemaphoreType.DMA,
            pltpu.SemaphoreType.DMA,
            pltpu.SemaphoreType.DMA,
            pltpu.SemaphoreType.DMA,
            pltpu.SemaphoreType.DMA,
        ],
    )
    def k(table_hbm, tok_hbm, out_hbm, idx_v, buf_v,
          g0, g1, g2, g3, s0, s1, s2, s3):
        wid = lax.axis_index("c") * NS + lax.axis_index("s")
        half = wid // NS
        c0 = wid * CPW

        pltpu.sync_copy(tok_hbm.at[wid], idx_v)
        bias = jnp.full((16,), half * 512, dtype=jnp.int32)
        for i in range(CPW):
            for q in range(8):
                sl = pl.ds(q * 16, 16)
                idx_v[i, sl] = idx_v[i, sl] + bias

        NSUB = 2 * CPW

        def src(s):
            i, kk = divmod(s, 2)
            return table_hbm.at[idx_v.at[i, pl.ds(kk * 64, 64)]]

        def dst(s):
            i, kk = divmod(s, 2)
            r = c0 + i - half * (L * BB)
            l = r // BB
            bb = r % BB
            return out_hbm.at[l, pl.ds(bb * 128 + kk * 64, 64),
                              pl.ds(half * HALF, HALF)]

        gsem = (g0, g1, g2, g3)
        ssem = (s0, s1, s2, s3)
        D = 4
        h_g = [None] * NSUB
        h_s = [None] * NSUB
        for s in range(D - 1):
            h_g[s] = pltpu.async_copy(src(s), buf_v.at[s % D], gsem[s % D])
        for s in range(NSUB):
            h_g[s].wait()
            h_s[s] = pltpu.async_copy(buf_v.at[s % D], dst(s), ssem[s % D])
            if s + D - 1 < NSUB:
                if s >= 1:
                    h_s[s - 1].wait()
                h_g[s + D - 1] = pltpu.async_copy(
                    src(s + D - 1), buf_v.at[(s + D - 1) % D],
                    gsem[(s + D - 1) % D])
        for s in range(NSUB - D, NSUB):
            h_s[s].wait()

    return k(table, tok)


def kernel(token, row_embed, col_embed):
    table = jnp.concatenate([row_embed, col_embed], axis=0)
    tok = token.astype(jnp.int32).transpose(2, 1, 0).reshape(NW, CPW, 128)
    out = _sc_gather(table, tok)
    return out.transpose(1, 0, 2)


# speedup vs baseline: 6.0292x; 1.0497x over previous
import jax
import jax.numpy as jnp
from jax import lax
from jax.experimental import pallas as pl
from jax.experimental.pallas import tpu as pltpu
from jax.experimental.pallas import tpu_sc as plsc

HALF = 384
B, L = 1024, 50
NC, NS = 2, 16
NW = NC * NS
BB = B // 128
CPW = 2 * L * BB // NW


def _sc_gather(table, tok):
    mesh = plsc.VectorSubcoreMesh(core_axis_name="c", subcore_axis_name="s")

    @pl.kernel(
        out_type=jax.ShapeDtypeStruct((L, B, 2 * HALF), jnp.float32),
        mesh=mesh,
        scratch_types=[
            pltpu.VMEM((CPW, 128), jnp.int32),
            pltpu.VMEM((2, 128, HALF), jnp.float32),
            pltpu.SemaphoreType.DMA,
            pltpu.SemaphoreType.DMA,
            pltpu.SemaphoreType.DMA,
            pltpu.SemaphoreType.DMA,
        ],
    )
    def k(table_hbm, tok_hbm, out_hbm, idx_v, buf_v, g0, g1, s0, s1):
        wid = lax.axis_index("c") * NS + lax.axis_index("s")
        half = wid // NS
        c0 = wid * CPW

        pltpu.sync_copy(tok_hbm.at[wid], idx_v)
        bias = jnp.full((16,), half * 512, dtype=jnp.int32)

        @pl.loop(0, CPW)
        def _bias(r):
            for q in range(8):
                sl = pl.ds(q * 16, 16)
                idx_v[r, sl] = idx_v[r, sl] + bias

        def start_gather(i, bb):
            return pltpu.async_copy(table_hbm.at[idx_v.at[i]],
                                    buf_v.at[bb], (g0, g1)[bb])

        def start_scatter(i, bb):
            r = c0 + i - half * (L * BB)
            l = r // BB
            blk = lax.rem(r, BB)
            return pltpu.async_copy(
                buf_v.at[bb],
                out_hbm.at[l, pl.ds(blk * 128, 128),
                           pl.ds(half * HALF, HALF)],
                (s0, s1)[bb])

        def wait_gather(bb):
            pltpu.make_async_copy(table_hbm.at[pl.ds(0, 128)],
                                  buf_v.at[bb], (g0, g1)[bb]).wait()

        def wait_scatter(bb):
            pltpu.make_async_copy(
                buf_v.at[bb],
                out_hbm.at[0, pl.ds(0, 128), pl.ds(0, HALF)],
                (s0, s1)[bb]).wait()

        start_gather(0, 0)
        start_gather(1, 1)
        wait_gather(0)
        start_scatter(0, 0)
        wait_scatter(0)
        start_gather(2, 0)
        wait_gather(1)
        start_scatter(1, 1)

        @pl.loop(2, CPW - 1, step=2)
        def _pipe(base):
            for t in range(2):
                i = base + t
                wait_scatter(1 - t)
                start_gather(i + 1, 1 - t)
                wait_gather(t)
                start_scatter(i, t)

        wait_scatter(1)
        wait_gather(0)
        start_scatter(CPW - 1, 0)
        wait_scatter(0)

    return k(table, tok)


def kernel(token, row_embed, col_embed):
    table = jnp.concatenate([row_embed, col_embed], axis=0)
    tok = token.astype(jnp.int32).transpose(2, 1, 0).reshape(NW, CPW, 128)
    out = _sc_gather(table, tok)
    return out.transpose(1, 0, 2)
